# baseline (speedup 1.0000x reference)
import functools

import jax
import jax.numpy as jnp
from jax import lax
from jax.experimental import pallas as pl
from jax.experimental.pallas import tpu as pltpu

H = 256
CE = 256


def _silu(x):
    return x * jax.nn.sigmoid(x)


def _edge_kernel(s_ref, eaT_ref, d4_ref, w1c_ref, w2_ref, b2_ref, pw1_ref,
                 pb1_ref, pw2_ref, pb2_ref, msg_ref, pw8_ref):
    s = s_ref[...]
    eaT = eaT_ref[...]
    ea_proj = lax.dot_general(eaT, w1c_ref[...], (((0,), (0,)), ((), ())),
                              preferred_element_type=jnp.float32)
    h1 = _silu(s + ea_proj)
    msg = _silu(jnp.dot(h1, w2_ref[...], preferred_element_type=jnp.float32)
                + b2_ref[...])
    ph = _silu(jnp.dot(msg, pw1_ref[...], preferred_element_type=jnp.float32)
               + pb1_ref[...])
    p = jnp.dot(ph, pw2_ref[...], preferred_element_type=jnp.float32) + pb2_ref[0, 0]
    msg_ref[...] = msg
    d4 = d4_ref[...]
    cnt = (lax.broadcasted_iota(jnp.int32, d4.shape, 1) == 3).astype(jnp.float32)
    pw8_ref[...] = p * d4 + cnt


def _edge_mlp(S, eaT, d4, w1c, w2, b2, pw1, pb1, pw2, pb2):
    E = S.shape[0]
    grid = E // CE
    return pl.pallas_call(
        _edge_kernel,
        grid=(grid,),
        in_specs=[
            pl.BlockSpec((CE, H), lambda i: (i, 0)),
            pl.BlockSpec((16, CE), lambda i: (0, i)),
            pl.BlockSpec((CE, 8), lambda i: (i, 0)),
            pl.BlockSpec((16, H), lambda i: (0, 0)),
            pl.BlockSpec((H, H), lambda i: (0, 0)),
            pl.BlockSpec((1, H), lambda i: (0, 0)),
            pl.BlockSpec((H, H), lambda i: (0, 0)),
            pl.BlockSpec((1, H), lambda i: (0, 0)),
            pl.BlockSpec((H, 1), lambda i: (0, 0)),
            pl.BlockSpec((1, 1), lambda i: (0, 0), memory_space=pltpu.SMEM),
        ],
        out_specs=[
            pl.BlockSpec((CE, H), lambda i: (i, 0)),
            pl.BlockSpec((CE, 8), lambda i: (i, 0)),
        ],
        out_shape=[
            jax.ShapeDtypeStruct((E, H), jnp.float32),
            jax.ShapeDtypeStruct((E, 8), jnp.float32),
        ],
    )(S, eaT, d4, w1c, w2, b2, pw1, pb1, pw2, pb2)


def _node_pre_kernel(nf_ref, w1a_ref, b1_ref, w1b_ref, a_ref, b_ref):
    nf = nf_ref[...]
    a_ref[...] = jnp.dot(nf, w1a_ref[...], preferred_element_type=jnp.float32) + b1_ref[...]
    b_ref[...] = jnp.dot(nf, w1b_ref[...], preferred_element_type=jnp.float32)


def _node_pre(nf, w1a, b1, w1b, bn):
    n = nf.shape[0]
    grid = n // bn
    return pl.pallas_call(
        _node_pre_kernel,
        grid=(grid,),
        in_specs=[
            pl.BlockSpec((bn, H), lambda i: (i, 0)),
            pl.BlockSpec((H, H), lambda i: (0, 0)),
            pl.BlockSpec((1, H), lambda i: (0, 0)),
            pl.BlockSpec((H, H), lambda i: (0, 0)),
        ],
        out_specs=[
            pl.BlockSpec((bn, H), lambda i: (i, 0)),
            pl.BlockSpec((bn, H), lambda i: (i, 0)),
        ],
        out_shape=[
            jax.ShapeDtypeStruct((n, H), jnp.float32),
            jax.ShapeDtypeStruct((n, H), jnp.float32),
        ],
    )(nf, w1a, b1, w1b)


def _node_post_kernel(nf_ref, mlo_ref, mhi_ref, ap_ref, pos_ref, vel_ref,
                      nfw1_ref, nfb1_ref, nfw2_ref, nfb2_ref,
                      vw1_ref, vb1_ref, vw2_ref, vb2_ref,
                      newf_ref, newp_ref):
    nf = nf_ref[...]
    ap = ap_ref[...]
    cnt = jnp.maximum(ap[:, 3:4], 1.0)
    magg = jnp.concatenate([mlo_ref[...], mhi_ref[...]], axis=1) / cnt
    nf2_w1 = (jnp.dot(nf, nfw1_ref[0], preferred_element_type=jnp.float32)
              + jnp.dot(magg, nfw1_ref[1], preferred_element_type=jnp.float32))
    hh = _silu(nf2_w1 + nfb1_ref[...])
    newf_ref[...] = jnp.dot(hh, nfw2_ref[...], preferred_element_type=jnp.float32) + nfb2_ref[...]
    vh = _silu(jnp.dot(nf, vw1_ref[...], preferred_element_type=jnp.float32) + vb1_ref[...])
    vf = jnp.dot(vh, vw2_ref[...], preferred_element_type=jnp.float32) + vb2_ref[0, 0]
    newp_ref[...] = pos_ref[...] + ap / cnt + vf * vel_ref[...]


def _node_post(nf, mlo, mhi, accp, pos8, vel8, nfW1, nfb1, nfW2, nfb2,
               vW1, vb1, vW2, vb2, bn):
    n = nf.shape[0]
    grid = n // bn
    nfW1s = nfW1.reshape(2, H, H)
    return pl.pallas_call(
        _node_post_kernel,
        grid=(grid,),
        in_specs=[
            pl.BlockSpec((bn, H), lambda i: (i, 0)),
            pl.BlockSpec((bn, H // 2), lambda i: (i, 0)),
            pl.BlockSpec((bn, H // 2), lambda i: (i, 0)),
            pl.BlockSpec((bn, 8), lambda i: (i, 0)),
            pl.BlockSpec((bn, 8), lambda i: (i, 0)),
            pl.BlockSpec((bn, 8), lambda i: (i, 0)),
            pl.BlockSpec((2, H, H), lambda i: (0, 0, 0)),
            pl.BlockSpec((1, H), lambda i: (0, 0)),
            pl.BlockSpec((H, H), lambda i: (0, 0)),
            pl.BlockSpec((1, H), lambda i: (0, 0)),
            pl.BlockSpec((H, H), lambda i: (0, 0)),
            pl.BlockSpec((1, H), lambda i: (0, 0)),
            pl.BlockSpec((H, 1), lambda i: (0, 0)),
            pl.BlockSpec((1, 1), lambda i: (0, 0), memory_space=pltpu.SMEM),
        ],
        out_specs=[
            pl.BlockSpec((bn, H), lambda i: (i, 0)),
            pl.BlockSpec((bn, 8), lambda i: (i, 0)),
        ],
        out_shape=[
            jax.ShapeDtypeStruct((n, H), jnp.float32),
            jax.ShapeDtypeStruct((n, 8), jnp.float32),
        ],
    )(nf, mlo, mhi, accp, pos8, vel8, nfW1s, nfb1, nfW2, nfb2, vW1, vb1, vW2, vb2)


def kernel(node_feat, node_pos, node_vel, edge_index, edge_attr, msg_W1,
           msg_b1, msg_W2, msg_b2, pos_W1, pos_b1, pos_W2, pos_b2, nf_W1,
           nf_b1, nf_W2, nf_b2, vel_W1, vel_b1, vel_W2, vel_b2):
    n = node_feat.shape[0]
    e = edge_index.shape[1]
    row = edge_index[0]
    col = edge_index[1]

    bn = 256
    n_pad = ((n + bn - 1) // bn) * bn
    nfp = jnp.pad(node_feat, ((0, n_pad - n), (0, 0)))

    A, B = _node_pre(nfp, msg_W1[:H], msg_b1[None, :], msg_W1[H:2 * H], bn)

    diff = node_pos[row] - node_pos[col]
    dist = jnp.sum(diff * diff, axis=-1, keepdims=True)
    S = A[row] + B[col] + dist * msg_W1[2 * H + 16][None, :]
    d4 = jnp.pad(diff, ((0, 0), (0, 5)))

    eaT = edge_attr.T
    msg, pw8 = _edge_mlp(S, eaT, d4, msg_W1[2 * H:2 * H + 16],
                         msg_W2, msg_b2[None, :], pos_W1, pos_b1[None, :],
                         pos_W2, pos_b2.reshape(1, 1))

    acc_lo = jax.ops.segment_sum(msg[:, :H // 2], row, num_segments=n_pad)
    acc_hi = jax.ops.segment_sum(msg[:, H // 2:], row, num_segments=n_pad)
    acc_p = jax.ops.segment_sum(pw8, row, num_segments=n_pad)

    pos8 = jnp.pad(node_pos, ((0, n_pad - n), (0, 5)))
    vel8 = jnp.pad(node_vel, ((0, n_pad - n), (0, 5)))
    newf, newp8 = _node_post(nfp, acc_lo, acc_hi, acc_p, pos8, vel8,
                             nf_W1, nf_b1[None, :], nf_W2, nf_b2[None, :],
                             vel_W1, vel_b1[None, :], vel_W2,
                             vel_b2.reshape(1, 1), bn)
    return (newf[:n], newp8[:n, :3])


# speedup vs baseline: 1.5048x; 1.5048x over previous
import functools

import jax
import jax.numpy as jnp
from jax import lax
from jax.experimental import pallas as pl
from jax.experimental.pallas import tpu as pltpu
from jax.experimental.pallas import tpu_sc as plsc

H = 256
CE = 256
CHUNK = 128
NTILES = 32


def _silu(x):
    return x * jax.nn.sigmoid(x)


def _make_gather_kernel(n_pad, e_pad):
    per_tile = e_pad // NTILES
    n_chunks = per_tile // CHUNK
    mesh = plsc.VectorSubcoreMesh(core_axis_name="c", subcore_axis_name="s")

    @functools.partial(
        pl.kernel,
        out_type=[
            jax.ShapeDtypeStruct((e_pad, H), jnp.float32),
            jax.ShapeDtypeStruct((e_pad * 8,), jnp.float32),
        ],
        mesh=mesh,
        compiler_params=pltpu.CompilerParams(needs_layout_passes=False),
        scratch_types=[
            pltpu.VMEM((CHUNK, H), jnp.float32),
            pltpu.VMEM((CHUNK, H), jnp.float32),
            pltpu.VMEM((CHUNK,), jnp.int32),
            pltpu.VMEM((CHUNK,), jnp.int32),
            pltpu.VMEM((n_pad,), jnp.float32),
            pltpu.VMEM((n_pad,), jnp.float32),
            pltpu.VMEM((n_pad,), jnp.float32),
            pltpu.VMEM((CHUNK * 8,), jnp.float32),
            pltpu.SemaphoreType.DMA,
            pltpu.SemaphoreType.DMA,
        ],
    )
    def k2(a_hbm, b_hbm, row_hbm, col_hbm, px_hbm, py_hbm, pz_hbm,
           s_hbm, d4_hbm, ar, br, rowi, coli, px, py, pz, d4b, semA, semB):
        wid = lax.axis_index("s") * 2 + lax.axis_index("c")
        pltpu.sync_copy(px_hbm, px)
        pltpu.sync_copy(py_hbm, py)
        pltpu.sync_copy(pz_hbm, pz)
        zero16 = jnp.zeros((16,), jnp.float32)

        def _zero(i, _):
            d4b[pl.ds(i * 16, 16)] = zero16
            return 0

        lax.fori_loop(0, CHUNK * 8 // 16, _zero, 0)
        lane = lax.iota(jnp.int32, 16)

        def body(it, _):
            base = wid * per_tile + it * CHUNK
            pltpu.sync_copy(row_hbm.at[pl.ds(base, CHUNK)], rowi)
            pltpu.sync_copy(col_hbm.at[pl.ds(base, CHUNK)], coli)
            cpa = pltpu.async_copy(a_hbm.at[rowi], ar, semA)
            cpb = pltpu.async_copy(b_hbm.at[coli], br, semB)
            for j in range(CHUNK // 16):
                r16 = rowi[pl.ds(j * 16, 16)]
                c16 = coli[pl.ds(j * 16, 16)]
                dx = (plsc.load_gather(px, [r16]) - plsc.load_gather(px, [c16]))
                dy = (plsc.load_gather(py, [r16]) - plsc.load_gather(py, [c16]))
                dz = (plsc.load_gather(pz, [r16]) - plsc.load_gather(pz, [c16]))
                flat = lane * 8 + (j * CHUNK)
                plsc.store_scatter(d4b, [flat], dx)
                plsc.store_scatter(d4b, [flat + 1], dy)
                plsc.store_scatter(d4b, [flat + 2], dz)
            cpa.wait()
            cpb.wait()

            def add_row(r, _):
                for f in range(H // 16):
                    sl = pl.ds(f * 16, 16)
                    ar[r, sl] = ar[r, sl] + br[r, sl]
                return 0

            lax.fori_loop(0, CHUNK, add_row, 0)
            pltpu.sync_copy(ar, s_hbm.at[pl.ds(base, CHUNK), :])
            pltpu.sync_copy(d4b, d4_hbm.at[pl.ds(base * 8, CHUNK * 8)])
            return 0

        lax.fori_loop(0, n_chunks, body, 0)

    return k2


def _edge_kernel(s_ref, eaT_ref, d4_ref, w1c_ref, w1d_ref, w2_ref, b2_ref,
                 pw1_ref, pb1_ref, pw2_ref, pb2_ref, msg_ref, pw8_ref):
    s = s_ref[...]
    eaT = eaT_ref[...]
    d4 = d4_ref[...]
    ea_proj = lax.dot_general(eaT, w1c_ref[...], (((0,), (0,)), ((), ())),
                              preferred_element_type=jnp.float32)
    dist = jnp.sum(d4 * d4, axis=1, keepdims=True)
    h1 = _silu(s + ea_proj + dist * w1d_ref[...])
    msg = _silu(jnp.dot(h1, w2_ref[...], preferred_element_type=jnp.float32)
                + b2_ref[...])
    ph = _silu(jnp.dot(msg, pw1_ref[...], preferred_element_type=jnp.float32)
               + pb1_ref[...])
    p = jnp.dot(ph, pw2_ref[...], preferred_element_type=jnp.float32) + pb2_ref[0, 0]
    msg_ref[...] = msg
    cnt = (lax.broadcasted_iota(jnp.int32, d4.shape, 1) == 3).astype(jnp.float32)
    pw8_ref[...] = p * d4 + cnt


def _edge_mlp(S, eaT, d4, w1c, w1d, w2, b2, pw1, pb1, pw2, pb2):
    E = S.shape[0]
    grid = E // CE
    return pl.pallas_call(
        _edge_kernel,
        grid=(grid,),
        in_specs=[
            pl.BlockSpec((CE, H), lambda i: (i, 0)),
            pl.BlockSpec((16, CE), lambda i: (0, i)),
            pl.BlockSpec((CE, 8), lambda i: (i, 0)),
            pl.BlockSpec((16, H), lambda i: (0, 0)),
            pl.BlockSpec((1, H), lambda i: (0, 0)),
            pl.BlockSpec((H, H), lambda i: (0, 0)),
            pl.BlockSpec((1, H), lambda i: (0, 0)),
            pl.BlockSpec((H, H), lambda i: (0, 0)),
            pl.BlockSpec((1, H), lambda i: (0, 0)),
            pl.BlockSpec((H, 1), lambda i: (0, 0)),
            pl.BlockSpec((1, 1), lambda i: (0, 0), memory_space=pltpu.SMEM),
        ],
        out_specs=[
            pl.BlockSpec((CE, H), lambda i: (i, 0)),
            pl.BlockSpec((CE, 8), lambda i: (i, 0)),
        ],
        out_shape=[
            jax.ShapeDtypeStruct((E, H), jnp.float32),
            jax.ShapeDtypeStruct((E, 8), jnp.float32),
        ],
    )(S, eaT, d4, w1c, w1d, w2, b2, pw1, pb1, pw2, pb2)


def _node_pre_kernel(nf_ref, w1a_ref, b1_ref, w1b_ref, a_ref, b_ref):
    nf = nf_ref[...]
    a_ref[...] = jnp.dot(nf, w1a_ref[...], preferred_element_type=jnp.float32) + b1_ref[...]
    b_ref[...] = jnp.dot(nf, w1b_ref[...], preferred_element_type=jnp.float32)


def _node_pre(nf, w1a, b1, w1b, bn):
    n = nf.shape[0]
    grid = n // bn
    return pl.pallas_call(
        _node_pre_kernel,
        grid=(grid,),
        in_specs=[
            pl.BlockSpec((bn, H), lambda i: (i, 0)),
            pl.BlockSpec((H, H), lambda i: (0, 0)),
            pl.BlockSpec((1, H), lambda i: (0, 0)),
            pl.BlockSpec((H, H), lambda i: (0, 0)),
        ],
        out_specs=[
            pl.BlockSpec((bn, H), lambda i: (i, 0)),
            pl.BlockSpec((bn, H), lambda i: (i, 0)),
        ],
        out_shape=[
            jax.ShapeDtypeStruct((n, H), jnp.float32),
            jax.ShapeDtypeStruct((n, H), jnp.float32),
        ],
    )(nf, w1a, b1, w1b)


def _node_post_kernel(nf_ref, mlo_ref, mhi_ref, ap_ref, pos_ref, vel_ref,
                      nfw1_ref, nfb1_ref, nfw2_ref, nfb2_ref,
                      vw1_ref, vb1_ref, vw2_ref, vb2_ref,
                      newf_ref, newp_ref):
    nf = nf_ref[...]
    ap = ap_ref[...]
    cnt = jnp.maximum(ap[:, 3:4], 1.0)
    magg = jnp.concatenate([mlo_ref[...], mhi_ref[...]], axis=1) / cnt
    nf2_w1 = (jnp.dot(nf, nfw1_ref[0], preferred_element_type=jnp.float32)
              + jnp.dot(magg, nfw1_ref[1], preferred_element_type=jnp.float32))
    hh = _silu(nf2_w1 + nfb1_ref[...])
    newf_ref[...] = jnp.dot(hh, nfw2_ref[...], preferred_element_type=jnp.float32) + nfb2_ref[...]
    vh = _silu(jnp.dot(nf, vw1_ref[...], preferred_element_type=jnp.float32) + vb1_ref[...])
    vf = jnp.dot(vh, vw2_ref[...], preferred_element_type=jnp.float32) + vb2_ref[0, 0]
    newp_ref[...] = pos_ref[...] + ap / cnt + vf * vel_ref[...]


def _node_post(nf, mlo, mhi, accp, pos8, vel8, nfW1, nfb1, nfW2, nfb2,
               vW1, vb1, vW2, vb2, bn):
    n = nf.shape[0]
    grid = n // bn
    nfW1s = nfW1.reshape(2, H, H)
    return pl.pallas_call(
        _node_post_kernel,
        grid=(grid,),
        in_specs=[
            pl.BlockSpec((bn, H), lambda i: (i, 0)),
            pl.BlockSpec((bn, H // 2), lambda i: (i, 0)),
            pl.BlockSpec((bn, H // 2), lambda i: (i, 0)),
            pl.BlockSpec((bn, 8), lambda i: (i, 0)),
            pl.BlockSpec((bn, 8), lambda i: (i, 0)),
            pl.BlockSpec((bn, 8), lambda i: (i, 0)),
            pl.BlockSpec((2, H, H), lambda i: (0, 0, 0)),
            pl.BlockSpec((1, H), lambda i: (0, 0)),
            pl.BlockSpec((H, H), lambda i: (0, 0)),
            pl.BlockSpec((1, H), lambda i: (0, 0)),
            pl.BlockSpec((H, H), lambda i: (0, 0)),
            pl.BlockSpec((1, H), lambda i: (0, 0)),
            pl.BlockSpec((H, 1), lambda i: (0, 0)),
            pl.BlockSpec((1, 1), lambda i: (0, 0), memory_space=pltpu.SMEM),
        ],
        out_specs=[
            pl.BlockSpec((bn, H), lambda i: (i, 0)),
            pl.BlockSpec((bn, 8), lambda i: (i, 0)),
        ],
        out_shape=[
            jax.ShapeDtypeStruct((n, H), jnp.float32),
            jax.ShapeDtypeStruct((n, 8), jnp.float32),
        ],
    )(nf, mlo, mhi, accp, pos8, vel8, nfW1s, nfb1, nfW2, nfb2, vW1, vb1, vW2, vb2)


def kernel(node_feat, node_pos, node_vel, edge_index, edge_attr, msg_W1,
           msg_b1, msg_W2, msg_b2, pos_W1, pos_b1, pos_W2, pos_b2, nf_W1,
           nf_b1, nf_W2, nf_b2, vel_W1, vel_b1, vel_W2, vel_b2):
    n = node_feat.shape[0]
    e = edge_index.shape[1]
    row = edge_index[0]
    col = edge_index[1]

    bn = 256
    n_pad = ((n + bn - 1) // bn) * bn
    step = NTILES * CHUNK
    e_pad = ((e + step - 1) // step) * step
    nfp = jnp.pad(node_feat, ((0, n_pad - n), (0, 0)))
    rowp = jnp.concatenate([row, jnp.full((e_pad - e,), n, jnp.int32)])
    colp = jnp.concatenate([col, jnp.full((e_pad - e,), n, jnp.int32)])

    A, B = _node_pre(nfp, msg_W1[:H], msg_b1[None, :], msg_W1[H:2 * H], bn)

    posp = jnp.pad(node_pos, ((0, n_pad - n), (0, 0)))
    k2 = _make_gather_kernel(n_pad, e_pad)
    S, d4f = k2(A, B, rowp, colp, posp[:, 0], posp[:, 1], posp[:, 2])
    d4 = d4f.reshape(e_pad, 8)

    eaT = jnp.pad(edge_attr.T, ((0, 0), (0, e_pad - e)))
    msg, pw8 = _edge_mlp(S, eaT, d4, msg_W1[2 * H:2 * H + 16],
                         msg_W1[2 * H + 16][None, :],
                         msg_W2, msg_b2[None, :], pos_W1, pos_b1[None, :],
                         pos_W2, pos_b2.reshape(1, 1))

    acc_lo = jax.ops.segment_sum(msg[:, :H // 2], rowp, num_segments=n_pad)
    acc_hi = jax.ops.segment_sum(msg[:, H // 2:], rowp, num_segments=n_pad)
    acc_p = jax.ops.segment_sum(pw8, rowp, num_segments=n_pad)

    pos8 = jnp.pad(node_pos, ((0, n_pad - n), (0, 5)))
    vel8 = jnp.pad(node_vel, ((0, n_pad - n), (0, 5)))
    newf, newp8 = _node_post(nfp, acc_lo, acc_hi, acc_p, pos8, vel8,
                             nf_W1, nf_b1[None, :], nf_W2, nf_b2[None, :],
                             vel_W1, vel_b1[None, :], vel_W2,
                             vel_b2.reshape(1, 1), bn)
    return (newf[:n], newp8[:n, :3])


# speedup vs baseline: 2.1624x; 1.4370x over previous
import functools

import jax
import jax.numpy as jnp
from jax import lax
from jax.experimental import pallas as pl
from jax.experimental.pallas import tpu as pltpu
from jax.experimental.pallas import tpu_sc as plsc

H = 256
HH = 128
CE = 512
CHUNK = 128
NTILES = 32


def _silu(x):
    return x * jax.nn.sigmoid(x)


def _node_pre_kernel(nf_ref, w1_ref, b1_ref, alo_ref, ahi_ref, blo_ref,
                     bhi_ref):
    nf = nf_ref[...].astype(jnp.bfloat16)
    w = w1_ref[...].astype(jnp.bfloat16)
    a = jnp.dot(nf, w[0], preferred_element_type=jnp.float32) + b1_ref[...]
    b = jnp.dot(nf, w[1], preferred_element_type=jnp.float32)
    alo_ref[...] = a[:, :HH]
    ahi_ref[...] = a[:, HH:]
    blo_ref[...] = b[:, :HH]
    bhi_ref[...] = b[:, HH:]


def _node_pre(nf, w12, b1, bn):
    n = nf.shape[0]
    grid = n // bn
    outs = [jax.ShapeDtypeStruct((n, HH), jnp.float32)] * 4
    return pl.pallas_call(
        _node_pre_kernel,
        grid=(grid,),
        in_specs=[
            pl.BlockSpec((bn, H), lambda i: (i, 0)),
            pl.BlockSpec((2, H, H), lambda i: (0, 0, 0)),
            pl.BlockSpec((1, H), lambda i: (0, 0)),
        ],
        out_specs=[pl.BlockSpec((bn, HH), lambda i: (i, 0))] * 4,
        out_shape=outs,
    )(nf, w12, b1)


def _make_gather_kernel(n_pad, e_pad):
    per_tile = e_pad // NTILES
    n_chunks = per_tile // CHUNK
    mesh = plsc.VectorSubcoreMesh(core_axis_name="c", subcore_axis_name="s")

    @functools.partial(
        pl.kernel,
        out_type=[
            jax.ShapeDtypeStruct((e_pad, HH), jnp.float32),
            jax.ShapeDtypeStruct((e_pad, HH), jnp.float32),
            jax.ShapeDtypeStruct((e_pad * 8,), jnp.float32),
        ],
        mesh=mesh,
        compiler_params=pltpu.CompilerParams(needs_layout_passes=False,
                                             use_tc_tiling_on_sc=False),
        scratch_types=[
            pltpu.VMEM((CHUNK, HH), jnp.float32),
            pltpu.VMEM((CHUNK, HH), jnp.float32),
            pltpu.VMEM((CHUNK, HH), jnp.float32),
            pltpu.VMEM((CHUNK, HH), jnp.float32),
            pltpu.VMEM((CHUNK,), jnp.int32),
            pltpu.VMEM((CHUNK,), jnp.int32),
            pltpu.VMEM((n_pad,), jnp.float32),
            pltpu.VMEM((n_pad,), jnp.float32),
            pltpu.VMEM((n_pad,), jnp.float32),
            pltpu.VMEM((H,), jnp.float32),
            pltpu.VMEM((CHUNK,), jnp.float32),
            pltpu.VMEM((CHUNK * 8,), jnp.float32),
            pltpu.SemaphoreType.DMA,
            pltpu.SemaphoreType.DMA,
            pltpu.SemaphoreType.DMA,
            pltpu.SemaphoreType.DMA,
        ],
    )
    def k2(alo_hbm, ahi_hbm, blo_hbm, bhi_hbm, row_hbm, col_hbm,
           px_hbm, py_hbm, pz_hbm, w1d_hbm,
           slo_hbm, shi_hbm, d4_hbm,
           arl, arh, brl, brh, rowi, coli, px, py, pz, w1d, db, d4b,
           s0, s1, s2, s3):
        wid = lax.axis_index("s") * 2 + lax.axis_index("c")
        pltpu.sync_copy(px_hbm, px)
        pltpu.sync_copy(py_hbm, py)
        pltpu.sync_copy(pz_hbm, pz)
        pltpu.sync_copy(w1d_hbm, w1d)
        zero16 = jnp.zeros((16,), jnp.float32)

        def _zero(i, _):
            d4b[pl.ds(i * 16, 16)] = zero16
            return 0

        lax.fori_loop(0, CHUNK * 8 // 16, _zero, 0)
        lane = lax.iota(jnp.int32, 16)

        def body(it, _):
            base = wid * per_tile + it * CHUNK
            pltpu.sync_copy(row_hbm.at[pl.ds(base, CHUNK)], rowi)
            pltpu.sync_copy(col_hbm.at[pl.ds(base, CHUNK)], coli)
            cp0 = pltpu.async_copy(alo_hbm.at[rowi], arl, s0)
            cp1 = pltpu.async_copy(ahi_hbm.at[rowi], arh, s1)
            cp2 = pltpu.async_copy(blo_hbm.at[coli], brl, s2)
            cp3 = pltpu.async_copy(bhi_hbm.at[coli], brh, s3)
            for j in range(CHUNK // 16):
                r16 = rowi[pl.ds(j * 16, 16)]
                c16 = coli[pl.ds(j * 16, 16)]
                dx = (plsc.load_gather(px, [r16]) - plsc.load_gather(px, [c16]))
                dy = (plsc.load_gather(py, [r16]) - plsc.load_gather(py, [c16]))
                dz = (plsc.load_gather(pz, [r16]) - plsc.load_gather(pz, [c16]))
                db[pl.ds(j * 16, 16)] = dx * dx + dy * dy + dz * dz
                flat = lane * 8 + (j * CHUNK)
                plsc.store_scatter(d4b, [flat], dx)
                plsc.store_scatter(d4b, [flat + 1], dy)
                plsc.store_scatter(d4b, [flat + 2], dz)
            cp0.wait()
            cp1.wait()
            cp2.wait()
            cp3.wait()

            def add_row(r, _):
                d = plsc.load_gather(db, [jnp.full((16,), r, jnp.int32)])
                for f in range(HH // 16):
                    sl = pl.ds(f * 16, 16)
                    arl[r, sl] = arl[r, sl] + brl[r, sl] + d * w1d[sl]
                for f in range(HH // 16):
                    sl = pl.ds(f * 16, 16)
                    sh = pl.ds(HH + f * 16, 16)
                    arh[r, sl] = arh[r, sl] + brh[r, sl] + d * w1d[sh]
                return 0

            lax.fori_loop(0, CHUNK, add_row, 0)
            pltpu.sync_copy(arl, slo_hbm.at[pl.ds(base, CHUNK), :])
            pltpu.sync_copy(arh, shi_hbm.at[pl.ds(base, CHUNK), :])
            pltpu.sync_copy(d4b, d4_hbm.at[pl.ds(base * 8, CHUNK * 8)])
            return 0

        lax.fori_loop(0, n_chunks, body, 0)

    return k2


def _edge_kernel(slo_ref, shi_ref, eaT_ref, w1c_ref, w2_ref, b2_ref,
                 pw1_ref, pb1_ref, pw2_ref, pb2_ref,
                 mlo_ref, mhi_ref, pb_ref):
    bf = jnp.bfloat16
    s = jnp.concatenate([slo_ref[...], shi_ref[...]], axis=1)
    ea_proj = lax.dot_general(eaT_ref[...].astype(bf),
                              w1c_ref[...].astype(bf),
                              (((0,), (0,)), ((), ())),
                              preferred_element_type=jnp.float32)
    h1 = _silu(s + ea_proj)
    msg = _silu(jnp.dot(h1.astype(bf), w2_ref[...].astype(bf),
                        preferred_element_type=jnp.float32) + b2_ref[...])
    ph = _silu(jnp.dot(msg.astype(bf), pw1_ref[...].astype(bf),
                       preferred_element_type=jnp.float32) + pb1_ref[...])
    p = jnp.dot(ph.astype(bf), pw2_ref[...].astype(bf),
                preferred_element_type=jnp.float32) + pb2_ref[0, 0]
    mlo_ref[...] = msg[:, :HH]
    mhi_ref[...] = msg[:, HH:]
    pb_ref[...] = jnp.broadcast_to(p, (p.shape[0], HH))


def _edge_mlp(slo, shi, eaT, w1c, w2, b2, pw1, pb1, pw2, pb2):
    E = slo.shape[0]
    grid = E // CE
    return pl.pallas_call(
        _edge_kernel,
        grid=(grid,),
        in_specs=[
            pl.BlockSpec((CE, HH), lambda i: (i, 0)),
            pl.BlockSpec((CE, HH), lambda i: (i, 0)),
            pl.BlockSpec((16, CE), lambda i: (0, i)),
            pl.BlockSpec((16, H), lambda i: (0, 0)),
            pl.BlockSpec((H, H), lambda i: (0, 0)),
            pl.BlockSpec((1, H), lambda i: (0, 0)),
            pl.BlockSpec((H, H), lambda i: (0, 0)),
            pl.BlockSpec((1, H), lambda i: (0, 0)),
            pl.BlockSpec((H, 1), lambda i: (0, 0)),
            pl.BlockSpec((1, 1), lambda i: (0, 0), memory_space=pltpu.SMEM),
        ],
        out_specs=[
            pl.BlockSpec((CE, HH), lambda i: (i, 0)),
            pl.BlockSpec((CE, HH), lambda i: (i, 0)),
            pl.BlockSpec((CE, HH), lambda i: (i, 0)),
        ],
        out_shape=[
            jax.ShapeDtypeStruct((E, HH), jnp.float32),
            jax.ShapeDtypeStruct((E, HH), jnp.float32),
            jax.ShapeDtypeStruct((E, HH), jnp.float32),
        ],
    )(slo, shi, eaT, w1c, w2, b2, pw1, pb1, pw2, pb2)


K4C = 64


def _make_scatter_kernel(n_pad, e_pad):
    per_tile = e_pad // 16
    n_chunks = per_tile // K4C
    rows_per_tile = n_pad // 16
    np16 = n_pad // 16
    prow_per_tile = np16 // 16
    mesh = plsc.VectorSubcoreMesh(core_axis_name="c", subcore_axis_name="s")

    @functools.partial(
        pl.kernel,
        out_type=[
            jax.ShapeDtypeStruct((n_pad, HH), jnp.float32),
            jax.ShapeDtypeStruct((n_pad, HH), jnp.float32),
            jax.ShapeDtypeStruct((np16, HH), jnp.float32),
        ],
        mesh=mesh,
        compiler_params=pltpu.CompilerParams(needs_layout_passes=False,
                                             use_tc_tiling_on_sc=False),
        scratch_types=[
            pltpu.VMEM((K4C, HH), jnp.float32),
            pltpu.VMEM((K4C, HH), jnp.float32),
            pltpu.VMEM((K4C, HH), jnp.float32),
            pltpu.VMEM((K4C, HH), jnp.float32),
            pltpu.VMEM((K4C * 8,), jnp.float32),
            pltpu.VMEM((K4C * 8,), jnp.float32),
            pltpu.VMEM((K4C,), jnp.int32),
            pltpu.VMEM((K4C,), jnp.int32),
            pltpu.VMEM((K4C,), jnp.int32),
            pltpu.VMEM((K4C,), jnp.int32),
            pltpu.VMEM((K4C, HH), jnp.float32),
            pltpu.VMEM_SHARED((n_pad, HH), jnp.float32),
            pltpu.VMEM_SHARED((n_pad // 16, HH), jnp.float32),
            pltpu.SemaphoreType.DMA,
            pltpu.SemaphoreType.DMA,
            pltpu.SemaphoreType.DMA,
            pltpu.SemaphoreType.DMA,
            pltpu.SemaphoreType.DMA,
            pltpu.SemaphoreType.DMA,
            pltpu.SemaphoreType.DMA,
            pltpu.SemaphoreType.DMA,
        ],
    )
    def k4(mlo_hbm, mhi_hbm, pb_hbm, d4_hbm, row_hbm, zero_hbm,
           out0_hbm, out1_hbm, outp_hbm,
           m0, m1, pb0, pb1, d40, d41, r0, r1, ri, pri, upd, accm, accp,
           sm0, sm1, sp0, sp1, sd0, sd1, sr0, sr1):
        cid = lax.axis_index("c")
        sid = lax.axis_index("s")
        mb = (m0, m1)
        pbb = (pb0, pb1)
        d4b = (d40, d41)
        rb = (r0, r1)
        sems_m = (sm0, sm1)
        sems_p = (sp0, sp1)
        sems_d = (sd0, sd1)
        sems_r = (sr0, sr1)
        rbase = sid * rows_per_tile
        rall = pl.ds(rbase, rows_per_tile)
        pltpu.sync_copy(zero_hbm.at[rall, :], accm.at[rall, :])
        pltpu.sync_copy(zero_hbm.at[pl.ds(sid * prow_per_tile, prow_per_tile), :],
                        accp.at[pl.ds(sid * prow_per_tile, prow_per_tile), :])
        zero16 = jnp.zeros((16,), jnp.float32)
        one16 = jnp.ones((16,), jnp.float32)
        lane = lax.iota(jnp.int32, 16)

        def _zupd(i, _):
            for f in range(HH // 16):
                upd[i, pl.ds(f * 16, 16)] = zero16
            return 0

        lax.fori_loop(0, K4C, _zupd, 0)
        plsc.subcore_barrier()

        def _issue(g, p2):
            base = sid * per_tile + g * K4C

            @pl.when(cid == 0)
            def _():
                pltpu.async_copy(mlo_hbm.at[pl.ds(base, K4C), :],
                                 mb[p2], sems_m[p2])
                pltpu.async_copy(pb_hbm.at[pl.ds(base, K4C), :],
                                 pbb[p2], sems_p[p2])
                pltpu.async_copy(d4_hbm.at[pl.ds(base * 8, K4C * 8)],
                                 d4b[p2], sems_d[p2])

            @pl.when(cid == 1)
            def _():
                pltpu.async_copy(mhi_hbm.at[pl.ds(base, K4C), :],
                                 mb[p2], sems_m[p2])

            pltpu.async_copy(row_hbm.at[pl.ds(base, K4C)], rb[p2],
                             sems_r[p2])

        def _wait(g, p2):
            base = sid * per_tile + g * K4C
            pltpu.make_async_copy(mlo_hbm.at[pl.ds(base, K4C), :],
                                  mb[p2], sems_m[p2]).wait()
            pltpu.make_async_copy(row_hbm.at[pl.ds(base, K4C)], rb[p2],
                                  sems_r[p2]).wait()

            @pl.when(cid == 0)
            def _():
                pltpu.make_async_copy(pb_hbm.at[pl.ds(base, K4C), :],
                                      pbb[p2], sems_p[p2]).wait()
                pltpu.make_async_copy(d4_hbm.at[pl.ds(base * 8, K4C * 8)],
                                      d4b[p2], sems_d[p2]).wait()

        _issue(0, 0)

        def body(g2, _):
            for p2 in range(2):
                g = g2 * 2 + p2
                _wait(g, p2)
                if p2 == 0:
                    _issue(g + 1, 1)
                else:
                    @pl.when(g2 < n_chunks // 2 - 1)
                    def _():
                        _issue(g + 1, 0)
                for f in range(K4C // 16):
                    sl = pl.ds(f * 16, 16)
                    ri[sl] = rb[p2][sl]
                pltpu.sync_copy(mb[p2], accm.at[ri], add=True)

                @pl.when(cid == 0)
                def _():
                    zero_i = jnp.zeros((16,), jnp.int32)
                    for j in range(K4C // 16):
                        e16 = lane + (j * 16)
                        r16 = rb[p2][pl.ds(j * 16, 16)]
                        pri[pl.ds(j * 16, 16)] = r16 >> 4
                        ln = (r16 & 15) * 8
                        p16 = plsc.load_gather(pbb[p2], [e16, zero_i])
                        for c in range(3):
                            d16 = plsc.load_gather(d4b[p2], [e16 * 8 + c])
                            plsc.store_scatter(upd, [e16, ln + c], p16 * d16)
                        plsc.store_scatter(upd, [e16, ln + 3], one16)
                    pltpu.sync_copy(upd, accp.at[pri], add=True)
                    for j in range(K4C // 16):
                        e16 = lane + (j * 16)
                        r16 = rb[p2][pl.ds(j * 16, 16)]
                        ln = (r16 & 15) * 8
                        for c in range(4):
                            plsc.store_scatter(upd, [e16, ln + c], zero16)
            return 0

        lax.fori_loop(0, n_chunks // 2, body, 0)
        plsc.subcore_barrier()

        @pl.when(cid == 0)
        def _():
            pltpu.sync_copy(accm.at[rall, :], out0_hbm.at[rall, :])
            psl = pl.ds(sid * prow_per_tile, prow_per_tile)
            pltpu.sync_copy(accp.at[psl, :], outp_hbm.at[psl, :])

        @pl.when(cid == 1)
        def _():
            pltpu.sync_copy(accm.at[rall, :], out1_hbm.at[rall, :])

    return k4


def _node_post_kernel(nf_ref, a0_ref, a1_ref, ap_ref, posp_ref, velp_ref,
                      nfw1_ref, nfb1_ref, nfw2_ref, nfb2_ref,
                      vw1_ref, vb1_ref, vw2_ref, vb2_ref,
                      newf_ref, newp_ref):
    bf = jnp.bfloat16
    bn = nf_ref.shape[0]
    nf = nf_ref[...]
    apk = ap_ref[...]
    io_r = lax.broadcasted_iota(jnp.int32, (bn, bn // 16), 0) // 16
    io_c = lax.broadcasted_iota(jnp.int32, (bn, bn // 16), 1)
    sel = (io_r == io_c).astype(bf)
    rep = jnp.dot(sel, apk.astype(bf), preferred_element_type=jnp.float32)
    li = lax.broadcasted_iota(jnp.int32, (bn, HH), 1)
    ni = lax.broadcasted_iota(jnp.int32, (bn, HH), 0)
    cmask = (li == (ni % 16) * 8 + 3).astype(jnp.float32)
    ones_col = jnp.ones((HH, 1), bf)
    cnt_col = jnp.dot((rep * cmask).astype(bf), ones_col,
                      preferred_element_type=jnp.float32)
    cnt = jnp.maximum(cnt_col, 1.0)

    magg = jnp.concatenate([a0_ref[...], a1_ref[...]], axis=1) / cnt
    nfw = nfw1_ref[...].astype(bf)
    nf2_w1 = (jnp.dot(nf.astype(bf), nfw[0], preferred_element_type=jnp.float32)
              + jnp.dot(magg.astype(bf), nfw[1],
                        preferred_element_type=jnp.float32))
    hh = _silu(nf2_w1 + nfb1_ref[...])
    newf_ref[...] = jnp.dot(hh.astype(bf), nfw2_ref[...].astype(bf),
                            preferred_element_type=jnp.float32) + nfb2_ref[...]

    vh = _silu(jnp.dot(nf.astype(bf), vw1_ref[...].astype(bf),
                       preferred_element_type=jnp.float32) + vb1_ref[...])
    vf = jnp.dot(vh.astype(bf), vw2_ref[...].astype(bf),
                 preferred_element_type=jnp.float32) + vb2_ref[0, 0]
    tmask = (li // 8 == ni % 16).astype(jnp.float32)
    selT = (io_r == io_c).astype(bf).T
    vf_p = jnp.dot(selT, (vf * tmask).astype(bf),
                   preferred_element_type=jnp.float32)
    inv_p = jnp.dot(selT, (tmask / cnt).astype(bf),
                    preferred_element_type=jnp.float32)
    newp_ref[...] = posp_ref[...] + apk * inv_p + vf_p * velp_ref[...]


def _node_post(nf, a0, a1, accp, posp, velp, nfW1, nfb1, nfW2, nfb2,
               vW1, vb1, vW2, vb2, bn):
    n = nf.shape[0]
    grid = n // bn
    nfW1s = nfW1.reshape(2, H, H)
    return pl.pallas_call(
        _node_post_kernel,
        grid=(grid,),
        in_specs=[
            pl.BlockSpec((bn, H), lambda i: (i, 0)),
            pl.BlockSpec((bn, HH), lambda i: (i, 0)),
            pl.BlockSpec((bn, HH), lambda i: (i, 0)),
            pl.BlockSpec((bn // 16, HH), lambda i: (i, 0)),
            pl.BlockSpec((bn // 16, HH), lambda i: (i, 0)),
            pl.BlockSpec((bn // 16, HH), lambda i: (i, 0)),
            pl.BlockSpec((2, H, H), lambda i: (0, 0, 0)),
            pl.BlockSpec((1, H), lambda i: (0, 0)),
            pl.BlockSpec((H, H), lambda i: (0, 0)),
            pl.BlockSpec((1, H), lambda i: (0, 0)),
            pl.BlockSpec((H, H), lambda i: (0, 0)),
            pl.BlockSpec((1, H), lambda i: (0, 0)),
            pl.BlockSpec((H, 1), lambda i: (0, 0)),
            pl.BlockSpec((1, 1), lambda i: (0, 0), memory_space=pltpu.SMEM),
        ],
        out_specs=[
            pl.BlockSpec((bn, H), lambda i: (i, 0)),
            pl.BlockSpec((bn // 16, HH), lambda i: (i, 0)),
        ],
        out_shape=[
            jax.ShapeDtypeStruct((n, H), jnp.float32),
            jax.ShapeDtypeStruct((n // 16, HH), jnp.float32),
        ],
    )(nf, a0, a1, accp, posp, velp, nfW1s, nfb1, nfW2, nfb2,
      vW1, vb1, vW2, vb2)


def kernel(node_feat, node_pos, node_vel, edge_index, edge_attr, msg_W1,
           msg_b1, msg_W2, msg_b2, pos_W1, pos_b1, pos_W2, pos_b2, nf_W1,
           nf_b1, nf_W2, nf_b2, vel_W1, vel_b1, vel_W2, vel_b2):
    n = node_feat.shape[0]
    e = edge_index.shape[1]
    row = edge_index[0]
    col = edge_index[1]

    bn = 256
    n_pad = ((n + bn - 1) // bn) * bn
    step = NTILES * CHUNK
    e_pad = ((e + step - 1) // step) * step
    nfp = jnp.pad(node_feat, ((0, n_pad - n), (0, 0)))
    rowp = jnp.concatenate([row, jnp.full((e_pad - e,), n, jnp.int32)])
    colp = jnp.concatenate([col, jnp.full((e_pad - e,), n, jnp.int32)])

    w12 = jnp.stack([msg_W1[:H], msg_W1[H:2 * H]])
    alo, ahi, blo, bhi = _node_pre(nfp, w12, msg_b1[None, :], bn)

    posp = jnp.pad(node_pos, ((0, n_pad - n), (0, 0)))
    k2 = _make_gather_kernel(n_pad, e_pad)
    slo, shi, d4f = k2(alo, ahi, blo, bhi, rowp, colp,
                       posp[:, 0], posp[:, 1], posp[:, 2],
                       msg_W1[2 * H + 16])

    eaT = jnp.pad(edge_attr.T, ((0, 0), (0, e_pad - e)))
    mlo, mhi, p_b = _edge_mlp(slo, shi, eaT, msg_W1[2 * H:2 * H + 16],
                              msg_W2, msg_b2[None, :], pos_W1,
                              pos_b1[None, :], pos_W2, pos_b2.reshape(1, 1))

    k4 = _make_scatter_kernel(n_pad, e_pad)
    zeros_acc = jnp.zeros((n_pad, HH), jnp.float32)
    a0, a1, accp = k4(mlo, mhi, p_b, d4f, rowp, zeros_acc)

    pos8 = jnp.pad(node_pos, ((0, n_pad - n), (0, 5))).reshape(n_pad // 16, HH)
    vel8 = jnp.pad(node_vel, ((0, n_pad - n), (0, 5))).reshape(n_pad // 16, HH)
    newf, newpP = _node_post(nfp, a0, a1, accp, pos8, vel8,
                             nf_W1, nf_b1[None, :], nf_W2, nf_b2[None, :],
                             vel_W1, vel_b1[None, :], vel_W2,
                             vel_b2.reshape(1, 1), bn)
    newp = newpP.reshape(n_pad, 8)[:n, :3]
    return (newf[:n], newp)


# speedup vs baseline: 2.8198x; 1.3040x over previous
import functools

import jax
import jax.numpy as jnp
from jax import lax
from jax.experimental import pallas as pl
from jax.experimental.pallas import tpu as pltpu
from jax.experimental.pallas import tpu_sc as plsc

H = 256
HH = 128
CE = 512
CHUNK = 128
NTILES = 32


def _silu(x):
    return x * jax.nn.sigmoid(x)


def _node_pre_kernel(nf_ref, w1_ref, b1_ref, alo_ref, ahi_ref, blo_ref,
                     bhi_ref):
    nf = nf_ref[...].astype(jnp.bfloat16)
    w = w1_ref[...].astype(jnp.bfloat16)
    a = jnp.dot(nf, w[0], preferred_element_type=jnp.float32) + b1_ref[...]
    b = jnp.dot(nf, w[1], preferred_element_type=jnp.float32)
    alo_ref[...] = a[:, :HH]
    ahi_ref[...] = a[:, HH:]
    blo_ref[...] = b[:, :HH]
    bhi_ref[...] = b[:, HH:]


def _node_pre(nf, w12, b1, bn):
    n = nf.shape[0]
    grid = n // bn
    outs = [jax.ShapeDtypeStruct((n, HH), jnp.float32)] * 4
    return pl.pallas_call(
        _node_pre_kernel,
        grid=(grid,),
        in_specs=[
            pl.BlockSpec((bn, H), lambda i: (i, 0)),
            pl.BlockSpec((2, H, H), lambda i: (0, 0, 0)),
            pl.BlockSpec((1, H), lambda i: (0, 0)),
        ],
        out_specs=[pl.BlockSpec((bn, HH), lambda i: (i, 0))] * 4,
        out_shape=outs,
    )(nf, w12, b1)


def _make_gather_kernel(n_pad, e_pad):
    per_tile = e_pad // NTILES
    n_chunks = per_tile // CHUNK
    mesh = plsc.VectorSubcoreMesh(core_axis_name="c", subcore_axis_name="s")

    @functools.partial(
        pl.kernel,
        out_type=[
            jax.ShapeDtypeStruct((e_pad, HH), jnp.float32),
            jax.ShapeDtypeStruct((e_pad, HH), jnp.float32),
            jax.ShapeDtypeStruct((e_pad * 8,), jnp.float32),
            jax.ShapeDtypeStruct((e_pad,), jnp.float32),
        ],
        mesh=mesh,
        compiler_params=pltpu.CompilerParams(needs_layout_passes=False,
                                             use_tc_tiling_on_sc=False),
        scratch_types=[
            pltpu.VMEM((CHUNK, HH), jnp.float32),
            pltpu.VMEM((CHUNK, HH), jnp.float32),
            pltpu.VMEM((CHUNK, HH), jnp.float32),
            pltpu.VMEM((CHUNK, HH), jnp.float32),
            pltpu.VMEM((CHUNK,), jnp.int32),
            pltpu.VMEM((CHUNK,), jnp.int32),
            pltpu.VMEM((n_pad,), jnp.float32),
            pltpu.VMEM((n_pad,), jnp.float32),
            pltpu.VMEM((n_pad,), jnp.float32),
            pltpu.VMEM((CHUNK,), jnp.float32),
            pltpu.VMEM((CHUNK * 8,), jnp.float32),
            pltpu.SemaphoreType.DMA,
            pltpu.SemaphoreType.DMA,
            pltpu.SemaphoreType.DMA,
            pltpu.SemaphoreType.DMA,
        ],
    )
    def k2(alo_hbm, ahi_hbm, blo_hbm, bhi_hbm, row_hbm, col_hbm,
           px_hbm, py_hbm, pz_hbm,
           slo_hbm, shi_hbm, d4_hbm, dist_hbm,
           arl, arh, brl, brh, rowi, coli, px, py, pz, db, d4b,
           s0, s1, s2, s3):
        wid = lax.axis_index("s") * 2 + lax.axis_index("c")
        pltpu.sync_copy(px_hbm, px)
        pltpu.sync_copy(py_hbm, py)
        pltpu.sync_copy(pz_hbm, pz)
        zero16 = jnp.zeros((16,), jnp.float32)

        def _zero(i, _):
            d4b[pl.ds(i * 16, 16)] = zero16
            return 0

        lax.fori_loop(0, CHUNK * 8 // 16, _zero, 0)
        lane = lax.iota(jnp.int32, 16)

        def body(it, _):
            base = wid * per_tile + it * CHUNK
            pltpu.sync_copy(row_hbm.at[pl.ds(base, CHUNK)], rowi)
            pltpu.sync_copy(col_hbm.at[pl.ds(base, CHUNK)], coli)
            cp0 = pltpu.async_copy(alo_hbm.at[rowi], arl, s0)
            cp1 = pltpu.async_copy(ahi_hbm.at[rowi], arh, s1)
            cp2 = pltpu.async_copy(blo_hbm.at[coli], brl, s2)
            cp3 = pltpu.async_copy(bhi_hbm.at[coli], brh, s3)
            for j in range(CHUNK // 16):
                r16 = rowi[pl.ds(j * 16, 16)]
                c16 = coli[pl.ds(j * 16, 16)]
                dx = (plsc.load_gather(px, [r16]) - plsc.load_gather(px, [c16]))
                dy = (plsc.load_gather(py, [r16]) - plsc.load_gather(py, [c16]))
                dz = (plsc.load_gather(pz, [r16]) - plsc.load_gather(pz, [c16]))
                db[pl.ds(j * 16, 16)] = dx * dx + dy * dy + dz * dz
                flat = lane * 8 + (j * CHUNK)
                plsc.store_scatter(d4b, [flat], dx)
                plsc.store_scatter(d4b, [flat + 1], dy)
                plsc.store_scatter(d4b, [flat + 2], dz)
            cp0.wait()
            cp1.wait()
            cp2.wait()
            cp3.wait()

            def add_row(r, _):
                for f in range(HH // 16):
                    sl = pl.ds(f * 16, 16)
                    arl[r, sl] = arl[r, sl] + brl[r, sl]
                    arh[r, sl] = arh[r, sl] + brh[r, sl]
                return 0

            lax.fori_loop(0, CHUNK, add_row, 0)
            pltpu.sync_copy(arl, slo_hbm.at[pl.ds(base, CHUNK), :])
            pltpu.sync_copy(arh, shi_hbm.at[pl.ds(base, CHUNK), :])
            pltpu.sync_copy(d4b, d4_hbm.at[pl.ds(base * 8, CHUNK * 8)])
            pltpu.sync_copy(db, dist_hbm.at[pl.ds(base, CHUNK)])
            return 0

        lax.fori_loop(0, n_chunks, body, 0)

    return k2


def _edge_kernel(slo_ref, shi_ref, eaT_ref, w1c_ref, w2_ref, b2_ref,
                 pw1_ref, pb1_ref, pw2_ref, pb2_ref,
                 mlo_ref, mhi_ref, pb_ref):
    bf = jnp.bfloat16
    s = jnp.concatenate([slo_ref[...], shi_ref[...]], axis=1)
    ea_proj = lax.dot_general(eaT_ref[...].astype(bf),
                              w1c_ref[...].astype(bf),
                              (((0,), (0,)), ((), ())),
                              preferred_element_type=jnp.float32)
    h1 = _silu(s + ea_proj)
    msg = _silu(jnp.dot(h1.astype(bf), w2_ref[...].astype(bf),
                        preferred_element_type=jnp.float32) + b2_ref[...])
    ph = _silu(jnp.dot(msg.astype(bf), pw1_ref[...].astype(bf),
                       preferred_element_type=jnp.float32) + pb1_ref[...])
    p = jnp.dot(ph.astype(bf), pw2_ref[...].astype(bf),
                preferred_element_type=jnp.float32) + pb2_ref[0, 0]
    mlo_ref[...] = msg[:, :HH]
    mhi_ref[...] = msg[:, HH:]
    pb_ref[...] = jnp.broadcast_to(p, (p.shape[0], HH))


def _edge_mlp(slo, shi, eaT, w1c, w2, b2, pw1, pb1, pw2, pb2):
    E = slo.shape[0]
    grid = E // CE
    return pl.pallas_call(
        _edge_kernel,
        grid=(grid,),
        in_specs=[
            pl.BlockSpec((CE, HH), lambda i: (i, 0)),
            pl.BlockSpec((CE, HH), lambda i: (i, 0)),
            pl.BlockSpec((24, CE), lambda i: (0, i)),
            pl.BlockSpec((24, H), lambda i: (0, 0)),
            pl.BlockSpec((H, H), lambda i: (0, 0)),
            pl.BlockSpec((1, H), lambda i: (0, 0)),
            pl.BlockSpec((H, H), lambda i: (0, 0)),
            pl.BlockSpec((1, H), lambda i: (0, 0)),
            pl.BlockSpec((H, 1), lambda i: (0, 0)),
            pl.BlockSpec((1, 1), lambda i: (0, 0), memory_space=pltpu.SMEM),
        ],
        out_specs=[
            pl.BlockSpec((CE, HH), lambda i: (i, 0)),
            pl.BlockSpec((CE, HH), lambda i: (i, 0)),
            pl.BlockSpec((CE, HH), lambda i: (i, 0)),
        ],
        out_shape=[
            jax.ShapeDtypeStruct((E, HH), jnp.float32),
            jax.ShapeDtypeStruct((E, HH), jnp.float32),
            jax.ShapeDtypeStruct((E, HH), jnp.float32),
        ],
    )(slo, shi, eaT, w1c, w2, b2, pw1, pb1, pw2, pb2)


K4C = 64


def _make_scatter_kernel(n_pad, e_pad):
    per_tile = e_pad // 16
    n_chunks = per_tile // K4C
    rows_per_tile = n_pad // 16
    np16 = n_pad // 16
    prow_per_tile = np16 // 16
    mesh = plsc.VectorSubcoreMesh(core_axis_name="c", subcore_axis_name="s")

    @functools.partial(
        pl.kernel,
        out_type=[
            jax.ShapeDtypeStruct((n_pad, HH), jnp.float32),
            jax.ShapeDtypeStruct((n_pad, HH), jnp.float32),
            jax.ShapeDtypeStruct((np16, HH), jnp.float32),
        ],
        mesh=mesh,
        compiler_params=pltpu.CompilerParams(needs_layout_passes=False,
                                             use_tc_tiling_on_sc=False),
        scratch_types=[
            pltpu.VMEM((K4C, HH), jnp.float32),
            pltpu.VMEM((K4C, HH), jnp.float32),
            pltpu.VMEM((K4C, 16), jnp.float32),
            pltpu.VMEM((K4C, 16), jnp.float32),
            pltpu.VMEM((K4C * 8,), jnp.float32),
            pltpu.VMEM((K4C * 8,), jnp.float32),
            pltpu.VMEM((K4C,), jnp.int32),
            pltpu.VMEM((K4C,), jnp.int32),
            pltpu.VMEM((K4C,), jnp.int32),
            pltpu.VMEM((K4C,), jnp.int32),
            pltpu.VMEM((K4C, HH), jnp.float32),
            pltpu.VMEM_SHARED((n_pad, HH), jnp.float32),
            pltpu.VMEM_SHARED((n_pad // 16, HH), jnp.float32),
            pltpu.SemaphoreType.DMA,
            pltpu.SemaphoreType.DMA,
            pltpu.SemaphoreType.DMA,
            pltpu.SemaphoreType.DMA,
            pltpu.SemaphoreType.DMA,
            pltpu.SemaphoreType.DMA,
            pltpu.SemaphoreType.DMA,
            pltpu.SemaphoreType.DMA,
        ],
    )
    def k4(mlo_hbm, mhi_hbm, pb_hbm, d4_hbm, row_hbm, zero_hbm,
           out0_hbm, out1_hbm, outp_hbm,
           m0, m1, pb0, pb1, d40, d41, r0, r1, ri, pri, upd, accm, accp,
           sm0, sm1, sp0, sp1, sd0, sd1, sr0, sr1):
        cid = lax.axis_index("c")
        sid = lax.axis_index("s")
        mb = (m0, m1)
        pbb = (pb0, pb1)
        d4b = (d40, d41)
        rb = (r0, r1)
        sems_m = (sm0, sm1)
        sems_p = (sp0, sp1)
        sems_d = (sd0, sd1)
        sems_r = (sr0, sr1)
        rbase = sid * rows_per_tile
        rall = pl.ds(rbase, rows_per_tile)
        pltpu.sync_copy(zero_hbm.at[rall, :], accm.at[rall, :])
        pltpu.sync_copy(zero_hbm.at[pl.ds(sid * prow_per_tile, prow_per_tile), :],
                        accp.at[pl.ds(sid * prow_per_tile, prow_per_tile), :])
        zero16 = jnp.zeros((16,), jnp.float32)
        one16 = jnp.ones((16,), jnp.float32)
        lane = lax.iota(jnp.int32, 16)

        def _zupd(i, _):
            for f in range(HH // 16):
                upd[i, pl.ds(f * 16, 16)] = zero16
            return 0

        lax.fori_loop(0, K4C, _zupd, 0)
        plsc.subcore_barrier()

        def _issue(g, p2):
            base = sid * per_tile + g * K4C

            @pl.when(cid == 0)
            def _():
                pltpu.async_copy(mlo_hbm.at[pl.ds(base, K4C), :],
                                 mb[p2], sems_m[p2])
                pltpu.async_copy(pb_hbm.at[pl.ds(base, K4C), pl.ds(0, 16)],
                                 pbb[p2], sems_p[p2])
                pltpu.async_copy(d4_hbm.at[pl.ds(base * 8, K4C * 8)],
                                 d4b[p2], sems_d[p2])

            @pl.when(cid == 1)
            def _():
                pltpu.async_copy(mhi_hbm.at[pl.ds(base, K4C), :],
                                 mb[p2], sems_m[p2])

            pltpu.async_copy(row_hbm.at[pl.ds(base, K4C)], rb[p2],
                             sems_r[p2])

        def _wait(g, p2):
            base = sid * per_tile + g * K4C
            pltpu.make_async_copy(mlo_hbm.at[pl.ds(base, K4C), :],
                                  mb[p2], sems_m[p2]).wait()
            pltpu.make_async_copy(row_hbm.at[pl.ds(base, K4C)], rb[p2],
                                  sems_r[p2]).wait()

            @pl.when(cid == 0)
            def _():
                pltpu.make_async_copy(pb_hbm.at[pl.ds(base, K4C), pl.ds(0, 16)],
                                      pbb[p2], sems_p[p2]).wait()
                pltpu.make_async_copy(d4_hbm.at[pl.ds(base * 8, K4C * 8)],
                                      d4b[p2], sems_d[p2]).wait()

        _issue(0, 0)

        def body(g2, _):
            for p2 in range(2):
                g = g2 * 2 + p2
                _wait(g, p2)
                if p2 == 0:
                    _issue(g + 1, 1)
                else:
                    @pl.when(g2 < n_chunks // 2 - 1)
                    def _():
                        _issue(g + 1, 0)
                for f in range(K4C // 16):
                    sl = pl.ds(f * 16, 16)
                    ri[sl] = rb[p2][sl]
                pltpu.sync_copy(mb[p2], accm.at[ri], add=True)

                @pl.when(cid == 0)
                def _():
                    zero_i = jnp.zeros((16,), jnp.int32)
                    for j in range(K4C // 16):
                        e16 = lane + (j * 16)
                        r16 = rb[p2][pl.ds(j * 16, 16)]
                        pri[pl.ds(j * 16, 16)] = r16 >> 4
                        ln = (r16 & 15) * 8
                        p16 = plsc.load_gather(pbb[p2], [e16, zero_i])
                        for c in range(3):
                            d16 = plsc.load_gather(d4b[p2], [e16 * 8 + c])
                            plsc.store_scatter(upd, [e16, ln + c], p16 * d16)
                        plsc.store_scatter(upd, [e16, ln + 3], one16)
                    pltpu.sync_copy(upd, accp.at[pri], add=True)
                    for j in range(K4C // 16):
                        e16 = lane + (j * 16)
                        r16 = rb[p2][pl.ds(j * 16, 16)]
                        ln = (r16 & 15) * 8
                        for c in range(4):
                            plsc.store_scatter(upd, [e16, ln + c], zero16)
            return 0

        lax.fori_loop(0, n_chunks // 2, body, 0)
        plsc.subcore_barrier()

        @pl.when(cid == 0)
        def _():
            pltpu.sync_copy(accm.at[rall, :], out0_hbm.at[rall, :])
            psl = pl.ds(sid * prow_per_tile, prow_per_tile)
            pltpu.sync_copy(accp.at[psl, :], outp_hbm.at[psl, :])

        @pl.when(cid == 1)
        def _():
            pltpu.sync_copy(accm.at[rall, :], out1_hbm.at[rall, :])

    return k4


def _node_post_kernel(nf_ref, a0_ref, a1_ref, ap_ref, posp_ref, velp_ref,
                      nfw1_ref, nfb1_ref, nfw2_ref, nfb2_ref,
                      vw1_ref, vb1_ref, vw2_ref, vb2_ref,
                      newf_ref, newp_ref):
    bf = jnp.bfloat16
    bn = nf_ref.shape[0]
    nf = nf_ref[...]
    apk = ap_ref[...]
    io_r = lax.broadcasted_iota(jnp.int32, (bn, bn // 16), 0) // 16
    io_c = lax.broadcasted_iota(jnp.int32, (bn, bn // 16), 1)
    sel = (io_r == io_c).astype(bf)
    rep = jnp.dot(sel, apk.astype(bf), preferred_element_type=jnp.float32)
    li = lax.broadcasted_iota(jnp.int32, (bn, HH), 1)
    ni = lax.broadcasted_iota(jnp.int32, (bn, HH), 0)
    cmask = (li == (ni % 16) * 8 + 3).astype(jnp.float32)
    ones_col = jnp.ones((HH, 1), bf)
    cnt_col = jnp.dot((rep * cmask).astype(bf), ones_col,
                      preferred_element_type=jnp.float32)
    cnt = jnp.maximum(cnt_col, 1.0)

    magg = jnp.concatenate([a0_ref[...], a1_ref[...]], axis=1) / cnt
    nfw = nfw1_ref[...].astype(bf)
    nf2_w1 = (jnp.dot(nf.astype(bf), nfw[0], preferred_element_type=jnp.float32)
              + jnp.dot(magg.astype(bf), nfw[1],
                        preferred_element_type=jnp.float32))
    hh = _silu(nf2_w1 + nfb1_ref[...])
    newf_ref[...] = jnp.dot(hh.astype(bf), nfw2_ref[...].astype(bf),
                            preferred_element_type=jnp.float32) + nfb2_ref[...]

    vh = _silu(jnp.dot(nf.astype(bf), vw1_ref[...].astype(bf),
                       preferred_element_type=jnp.float32) + vb1_ref[...])
    vf = jnp.dot(vh.astype(bf), vw2_ref[...].astype(bf),
                 preferred_element_type=jnp.float32) + vb2_ref[0, 0]
    tmask = (li // 8 == ni % 16).astype(jnp.float32)
    selT = (io_r == io_c).astype(bf).T
    vf_p = jnp.dot(selT, (vf * tmask).astype(bf),
                   preferred_element_type=jnp.float32)
    inv_p = jnp.dot(selT, (tmask / cnt).astype(bf),
                    preferred_element_type=jnp.float32)
    newp_ref[...] = posp_ref[...] + apk * inv_p + vf_p * velp_ref[...]


def _node_post(nf, a0, a1, accp, posp, velp, nfW1, nfb1, nfW2, nfb2,
               vW1, vb1, vW2, vb2, bn):
    n = nf.shape[0]
    grid = n // bn
    nfW1s = nfW1.reshape(2, H, H)
    return pl.pallas_call(
        _node_post_kernel,
        grid=(grid,),
        in_specs=[
            pl.BlockSpec((bn, H), lambda i: (i, 0)),
            pl.BlockSpec((bn, HH), lambda i: (i, 0)),
            pl.BlockSpec((bn, HH), lambda i: (i, 0)),
            pl.BlockSpec((bn // 16, HH), lambda i: (i, 0)),
            pl.BlockSpec((bn // 16, HH), lambda i: (i, 0)),
            pl.BlockSpec((bn // 16, HH), lambda i: (i, 0)),
            pl.BlockSpec((2, H, H), lambda i: (0, 0, 0)),
            pl.BlockSpec((1, H), lambda i: (0, 0)),
            pl.BlockSpec((H, H), lambda i: (0, 0)),
            pl.BlockSpec((1, H), lambda i: (0, 0)),
            pl.BlockSpec((H, H), lambda i: (0, 0)),
            pl.BlockSpec((1, H), lambda i: (0, 0)),
            pl.BlockSpec((H, 1), lambda i: (0, 0)),
            pl.BlockSpec((1, 1), lambda i: (0, 0), memory_space=pltpu.SMEM),
        ],
        out_specs=[
            pl.BlockSpec((bn, H), lambda i: (i, 0)),
            pl.BlockSpec((bn // 16, HH), lambda i: (i, 0)),
        ],
        out_shape=[
            jax.ShapeDtypeStruct((n, H), jnp.float32),
            jax.ShapeDtypeStruct((n // 16, HH), jnp.float32),
        ],
    )(nf, a0, a1, accp, posp, velp, nfW1s, nfb1, nfW2, nfb2,
      vW1, vb1, vW2, vb2)


def kernel(node_feat, node_pos, node_vel, edge_index, edge_attr, msg_W1,
           msg_b1, msg_W2, msg_b2, pos_W1, pos_b1, pos_W2, pos_b2, nf_W1,
           nf_b1, nf_W2, nf_b2, vel_W1, vel_b1, vel_W2, vel_b2):
    n = node_feat.shape[0]
    e = edge_index.shape[1]
    row = edge_index[0]
    col = edge_index[1]

    bn = 256
    n_pad = ((n + bn - 1) // bn) * bn
    step = NTILES * CHUNK
    e_pad = ((e + step - 1) // step) * step
    nfp = jnp.pad(node_feat, ((0, n_pad - n), (0, 0)))
    rowp = jnp.concatenate([row, jnp.full((e_pad - e,), n, jnp.int32)])
    colp = jnp.concatenate([col, jnp.full((e_pad - e,), n, jnp.int32)])

    w12 = jnp.stack([msg_W1[:H], msg_W1[H:2 * H]])
    alo, ahi, blo, bhi = _node_pre(nfp, w12, msg_b1[None, :], bn)

    posp = jnp.pad(node_pos, ((0, n_pad - n), (0, 0)))
    k2 = _make_gather_kernel(n_pad, e_pad)
    slo, shi, d4f, distE = k2(alo, ahi, blo, bhi, rowp, colp,
                              posp[:, 0], posp[:, 1], posp[:, 2])

    eaT = jnp.concatenate([
        jnp.pad(edge_attr.T, ((0, 0), (0, e_pad - e))),
        distE[None, :],
        jnp.zeros((7, e_pad), jnp.float32),
    ])
    w1c2 = jnp.concatenate([msg_W1[2 * H:2 * H + 17],
                            jnp.zeros((7, H), jnp.float32)])
    mlo, mhi, p_b = _edge_mlp(slo, shi, eaT, w1c2,
                              msg_W2, msg_b2[None, :], pos_W1,
                              pos_b1[None, :], pos_W2, pos_b2.reshape(1, 1))

    k4 = _make_scatter_kernel(n_pad, e_pad)
    zeros_acc = jnp.zeros((n_pad, HH), jnp.float32)
    a0, a1, accp = k4(mlo, mhi, p_b, d4f, rowp, zeros_acc)

    pos8 = jnp.pad(node_pos, ((0, n_pad - n), (0, 5))).reshape(n_pad // 16, HH)
    vel8 = jnp.pad(node_vel, ((0, n_pad - n), (0, 5))).reshape(n_pad // 16, HH)
    newf, newpP = _node_post(nfp, a0, a1, accp, pos8, vel8,
                             nf_W1, nf_b1[None, :], nf_W2, nf_b2[None, :],
                             vel_W1, vel_b1[None, :], vel_W2,
                             vel_b2.reshape(1, 1), bn)
    newp = newpP.reshape(n_pad, 8)[:n, :3]
    return (newf[:n], newp)


# speedup vs baseline: 3.2859x; 1.1653x over previous
import functools

import jax
import jax.numpy as jnp
from jax import lax
from jax.experimental import pallas as pl
from jax.experimental.pallas import tpu as pltpu
from jax.experimental.pallas import tpu_sc as plsc

H = 256
HH = 128
CE = 512
CHUNK = 128
NTILES = 32


def _silu(x):
    return x * jax.nn.sigmoid(x)


def _node_pre_kernel(nf_ref, w1_ref, b1_ref, alo_ref, ahi_ref, blo_ref,
                     bhi_ref):
    nf = nf_ref[...].astype(jnp.bfloat16)
    w = w1_ref[...].astype(jnp.bfloat16)
    a = jnp.dot(nf, w[0], preferred_element_type=jnp.float32) + b1_ref[...]
    b = jnp.dot(nf, w[1], preferred_element_type=jnp.float32)
    alo_ref[...] = a[:, :HH]
    ahi_ref[...] = a[:, HH:]
    blo_ref[...] = b[:, :HH]
    bhi_ref[...] = b[:, HH:]


def _node_pre(nf, w12, b1, bn):
    n = nf.shape[0]
    grid = n // bn
    outs = [jax.ShapeDtypeStruct((n, HH), jnp.float32)] * 4
    return pl.pallas_call(
        _node_pre_kernel,
        grid=(grid,),
        in_specs=[
            pl.BlockSpec((bn, H), lambda i: (i, 0)),
            pl.BlockSpec((2, H, H), lambda i: (0, 0, 0)),
            pl.BlockSpec((1, H), lambda i: (0, 0)),
        ],
        out_specs=[pl.BlockSpec((bn, HH), lambda i: (i, 0))] * 4,
        out_shape=outs,
    )(nf, w12, b1)


K2C = 64


def _make_gather_kernel(n_pad, e_pad):
    per_tile = e_pad // NTILES
    n_chunks = per_tile // K2C
    mesh = plsc.VectorSubcoreMesh(core_axis_name="c", subcore_axis_name="s")

    @functools.partial(
        pl.kernel,
        out_type=[
            jax.ShapeDtypeStruct((e_pad, HH), jnp.float32),
            jax.ShapeDtypeStruct((e_pad, HH), jnp.float32),
            jax.ShapeDtypeStruct((e_pad * 8,), jnp.float32),
            jax.ShapeDtypeStruct((e_pad,), jnp.float32),
        ],
        mesh=mesh,
        compiler_params=pltpu.CompilerParams(needs_layout_passes=False,
                                             use_tc_tiling_on_sc=False),
        scratch_types=(
            [pltpu.VMEM((K2C, HH), jnp.float32)] * 8 +
            [pltpu.VMEM((K2C,), jnp.int32)] * 4 +
            [pltpu.VMEM((n_pad,), jnp.float32)] * 3 +
            [pltpu.VMEM((K2C,), jnp.float32)] * 2 +
            [pltpu.VMEM((K2C * 8,), jnp.float32)] * 2 +
            [pltpu.SemaphoreType.DMA] * 20
        ),
    )
    def k2(alo_hbm, ahi_hbm, blo_hbm, bhi_hbm, row_hbm, col_hbm,
           px_hbm, py_hbm, pz_hbm,
           slo_hbm, shi_hbm, d4_hbm, dist_hbm,
           arl0, arh0, brl0, brh0, arl1, arh1, brl1, brh1,
           rowi0, coli0, rowi1, coli1, px, py, pz, db0, db1, d4b0, d4b1,
           sir0, sic0, sir1, sic1,
           sg00, sg01, sg02, sg03, sg10, sg11, sg12, sg13,
           sw00, sw01, sw02, sw03, sw10, sw11, sw12, sw13):
        wid = lax.axis_index("s") * 2 + lax.axis_index("c")
        pltpu.sync_copy(px_hbm, px)
        pltpu.sync_copy(py_hbm, py)
        pltpu.sync_copy(pz_hbm, pz)
        arl = (arl0, arl1)
        arh = (arh0, arh1)
        brl = (brl0, brl1)
        brh = (brh0, brh1)
        rowi = (rowi0, rowi1)
        coli = (coli0, coli1)
        db = (db0, db1)
        d4b = (d4b0, d4b1)
        sir = (sir0, sir1)
        sic = (sic0, sic1)
        sg = ((sg00, sg01, sg02, sg03), (sg10, sg11, sg12, sg13))
        sw = ((sw00, sw01, sw02, sw03), (sw10, sw11, sw12, sw13))
        zero16 = jnp.zeros((16,), jnp.float32)
        lane = lax.iota(jnp.int32, 16)

        def _zero(i, _):
            d4b0[pl.ds(i * 16, 16)] = zero16
            d4b1[pl.ds(i * 16, 16)] = zero16
            return 0

        lax.fori_loop(0, K2C * 8 // 16, _zero, 0)

        def base_of(g):
            return wid * per_tile + g * K2C

        def issue_idx(g, p):
            base = base_of(g)
            pltpu.async_copy(row_hbm.at[pl.ds(base, K2C)], rowi[p], sir[p])
            pltpu.async_copy(col_hbm.at[pl.ds(base, K2C)], coli[p], sic[p])

        def wait_idx(g, p):
            base = base_of(g)
            pltpu.make_async_copy(row_hbm.at[pl.ds(base, K2C)], rowi[p],
                                  sir[p]).wait()
            pltpu.make_async_copy(col_hbm.at[pl.ds(base, K2C)], coli[p],
                                  sic[p]).wait()

        def issue_gather(p):
            pltpu.async_copy(alo_hbm.at[rowi[p]], arl[p], sg[p][0])
            pltpu.async_copy(ahi_hbm.at[rowi[p]], arh[p], sg[p][1])
            pltpu.async_copy(blo_hbm.at[coli[p]], brl[p], sg[p][2])
            pltpu.async_copy(bhi_hbm.at[coli[p]], brh[p], sg[p][3])

        def wait_gather(p):
            pltpu.make_async_copy(alo_hbm.at[rowi[p]], arl[p], sg[p][0]).wait()
            pltpu.make_async_copy(ahi_hbm.at[rowi[p]], arh[p], sg[p][1]).wait()
            pltpu.make_async_copy(blo_hbm.at[coli[p]], brl[p], sg[p][2]).wait()
            pltpu.make_async_copy(bhi_hbm.at[coli[p]], brh[p], sg[p][3]).wait()

        def issue_writes(g, p):
            base = base_of(g)
            pltpu.async_copy(arl[p], slo_hbm.at[pl.ds(base, K2C), :], sw[p][0])
            pltpu.async_copy(arh[p], shi_hbm.at[pl.ds(base, K2C), :], sw[p][1])
            pltpu.async_copy(d4b[p], d4_hbm.at[pl.ds(base * 8, K2C * 8)],
                             sw[p][2])
            pltpu.async_copy(db[p], dist_hbm.at[pl.ds(base, K2C)], sw[p][3])

        def wait_writes(g, p):
            base = base_of(g)
            pltpu.make_async_copy(arl[p], slo_hbm.at[pl.ds(base, K2C), :],
                                  sw[p][0]).wait()
            pltpu.make_async_copy(arh[p], shi_hbm.at[pl.ds(base, K2C), :],
                                  sw[p][1]).wait()
            pltpu.make_async_copy(d4b[p], d4_hbm.at[pl.ds(base * 8, K2C * 8)],
                                  sw[p][2]).wait()
            pltpu.make_async_copy(db[p], dist_hbm.at[pl.ds(base, K2C)],
                                  sw[p][3]).wait()

        def pos_compute(p):
            for j in range(K2C // 16):
                r16 = rowi[p][pl.ds(j * 16, 16)]
                c16 = coli[p][pl.ds(j * 16, 16)]
                dx = (plsc.load_gather(px, [r16]) - plsc.load_gather(px, [c16]))
                dy = (plsc.load_gather(py, [r16]) - plsc.load_gather(py, [c16]))
                dz = (plsc.load_gather(pz, [r16]) - plsc.load_gather(pz, [c16]))
                db[p][pl.ds(j * 16, 16)] = dx * dx + dy * dy + dz * dz
                flat = lane * 8 + (j * 16) * 8
                plsc.store_scatter(d4b[p], [flat], dx)
                plsc.store_scatter(d4b[p], [flat + 1], dy)
                plsc.store_scatter(d4b[p], [flat + 2], dz)

        def add_rows(p):
            def add_row(r, _):
                for f in range(HH // 16):
                    sl = pl.ds(f * 16, 16)
                    arl[p][r, sl] = arl[p][r, sl] + brl[p][r, sl]
                    arh[p][r, sl] = arh[p][r, sl] + brh[p][r, sl]
                return 0

            lax.fori_loop(0, K2C, add_row, 0)

        issue_idx(0, 0)
        issue_idx(1, 1)
        wait_idx(0, 0)
        issue_gather(0)

        def body(g2, _):
            for p in range(2):
                g = g2 * 2 + p
                q = 1 - p
                pos_compute(p)
                if p == 0:
                    wait_idx(g + 1, q)

                    @pl.when(g2 > 0)
                    def _():
                        wait_writes(g - 1, q)

                    issue_gather(q)
                else:
                    @pl.when(g2 < n_chunks // 2 - 1)
                    def _():
                        wait_idx(g + 1, q)
                        wait_writes(g - 1, q)
                        issue_gather(q)
                wait_gather(p)
                add_rows(p)

                @pl.when(g2 < n_chunks // 2 - 1)
                def _():
                    issue_idx(g + 2, p)

                issue_writes(g, p)
            return 0

        lax.fori_loop(0, n_chunks // 2, body, 0)
        wait_writes(n_chunks - 2, 0)
        wait_writes(n_chunks - 1, 1)

    return k2


def _edge_kernel(slo_ref, shi_ref, eaT_ref, w1c_ref, w2_ref, b2_ref,
                 pw1_ref, pb1_ref, pw2_ref, pb2_ref,
                 mlo_ref, mhi_ref, pb_ref):
    bf = jnp.bfloat16
    s = jnp.concatenate([slo_ref[...], shi_ref[...]], axis=1)
    ea_proj = lax.dot_general(eaT_ref[...].astype(bf),
                              w1c_ref[...].astype(bf),
                              (((0,), (0,)), ((), ())),
                              preferred_element_type=jnp.float32)
    h1 = _silu(s + ea_proj)
    msg = _silu(jnp.dot(h1.astype(bf), w2_ref[...].astype(bf),
                        preferred_element_type=jnp.float32) + b2_ref[...])
    ph = _silu(jnp.dot(msg.astype(bf), pw1_ref[...].astype(bf),
                       preferred_element_type=jnp.float32) + pb1_ref[...])
    p = jnp.dot(ph.astype(bf), pw2_ref[...].astype(bf),
                preferred_element_type=jnp.float32) + pb2_ref[0, 0]
    mlo_ref[...] = msg[:, :HH]
    mhi_ref[...] = msg[:, HH:]
    pb_ref[...] = jnp.broadcast_to(p, (p.shape[0], HH))


def _edge_mlp(slo, shi, eaT, w1c, w2, b2, pw1, pb1, pw2, pb2):
    E = slo.shape[0]
    grid = E // CE
    return pl.pallas_call(
        _edge_kernel,
        grid=(grid,),
        in_specs=[
            pl.BlockSpec((CE, HH), lambda i: (i, 0)),
            pl.BlockSpec((CE, HH), lambda i: (i, 0)),
            pl.BlockSpec((24, CE), lambda i: (0, i)),
            pl.BlockSpec((24, H), lambda i: (0, 0)),
            pl.BlockSpec((H, H), lambda i: (0, 0)),
            pl.BlockSpec((1, H), lambda i: (0, 0)),
            pl.BlockSpec((H, H), lambda i: (0, 0)),
            pl.BlockSpec((1, H), lambda i: (0, 0)),
            pl.BlockSpec((H, 1), lambda i: (0, 0)),
            pl.BlockSpec((1, 1), lambda i: (0, 0), memory_space=pltpu.SMEM),
        ],
        out_specs=[
            pl.BlockSpec((CE, HH), lambda i: (i, 0)),
            pl.BlockSpec((CE, HH), lambda i: (i, 0)),
            pl.BlockSpec((CE, HH), lambda i: (i, 0)),
        ],
        out_shape=[
            jax.ShapeDtypeStruct((E, HH), jnp.float32),
            jax.ShapeDtypeStruct((E, HH), jnp.float32),
            jax.ShapeDtypeStruct((E, HH), jnp.float32),
        ],
    )(slo, shi, eaT, w1c, w2, b2, pw1, pb1, pw2, pb2)


K4C = 64


def _make_scatter_kernel(n_pad, e_pad):
    per_tile = e_pad // 16
    n_chunks = per_tile // K4C
    rows_per_tile = n_pad // 16
    np16 = n_pad // 16
    prow_per_tile = np16 // 16
    mesh = plsc.VectorSubcoreMesh(core_axis_name="c", subcore_axis_name="s")

    @functools.partial(
        pl.kernel,
        out_type=[
            jax.ShapeDtypeStruct((n_pad, HH), jnp.float32),
            jax.ShapeDtypeStruct((n_pad, HH), jnp.float32),
            jax.ShapeDtypeStruct((np16, HH), jnp.float32),
        ],
        mesh=mesh,
        compiler_params=pltpu.CompilerParams(needs_layout_passes=False,
                                             use_tc_tiling_on_sc=False),
        scratch_types=[
            pltpu.VMEM((K4C, HH), jnp.float32),
            pltpu.VMEM((K4C, HH), jnp.float32),
            pltpu.VMEM((K4C, 16), jnp.float32),
            pltpu.VMEM((K4C, 16), jnp.float32),
            pltpu.VMEM((K4C * 8,), jnp.float32),
            pltpu.VMEM((K4C * 8,), jnp.float32),
            pltpu.VMEM((K4C,), jnp.int32),
            pltpu.VMEM((K4C,), jnp.int32),
            pltpu.VMEM((K4C,), jnp.int32),
            pltpu.VMEM((K4C,), jnp.int32),
            pltpu.VMEM((K4C, HH), jnp.float32),
            pltpu.VMEM_SHARED((n_pad, HH), jnp.float32),
            pltpu.VMEM_SHARED((n_pad // 16, HH), jnp.float32),
            pltpu.SemaphoreType.DMA,
            pltpu.SemaphoreType.DMA,
            pltpu.SemaphoreType.DMA,
            pltpu.SemaphoreType.DMA,
            pltpu.SemaphoreType.DMA,
            pltpu.SemaphoreType.DMA,
            pltpu.SemaphoreType.DMA,
            pltpu.SemaphoreType.DMA,
        ],
    )
    def k4(mlo_hbm, mhi_hbm, pb_hbm, d4_hbm, row_hbm, zero_hbm,
           out0_hbm, out1_hbm, outp_hbm,
           m0, m1, pb0, pb1, d40, d41, r0, r1, ri, pri, upd, accm, accp,
           sm0, sm1, sp0, sp1, sd0, sd1, sr0, sr1):
        cid = lax.axis_index("c")
        sid = lax.axis_index("s")
        mb = (m0, m1)
        pbb = (pb0, pb1)
        d4b = (d40, d41)
        rb = (r0, r1)
        sems_m = (sm0, sm1)
        sems_p = (sp0, sp1)
        sems_d = (sd0, sd1)
        sems_r = (sr0, sr1)
        rbase = sid * rows_per_tile
        rall = pl.ds(rbase, rows_per_tile)
        pltpu.sync_copy(zero_hbm.at[rall, :], accm.at[rall, :])
        pltpu.sync_copy(zero_hbm.at[pl.ds(sid * prow_per_tile, prow_per_tile), :],
                        accp.at[pl.ds(sid * prow_per_tile, prow_per_tile), :])
        zero16 = jnp.zeros((16,), jnp.float32)
        one16 = jnp.ones((16,), jnp.float32)
        lane = lax.iota(jnp.int32, 16)

        def _zupd(i, _):
            for f in range(HH // 16):
                upd[i, pl.ds(f * 16, 16)] = zero16
            return 0

        lax.fori_loop(0, K4C, _zupd, 0)
        plsc.subcore_barrier()

        def _issue(g, p2):
            base = sid * per_tile + g * K4C

            @pl.when(cid == 0)
            def _():
                pltpu.async_copy(mlo_hbm.at[pl.ds(base, K4C), :],
                                 mb[p2], sems_m[p2])
                pltpu.async_copy(pb_hbm.at[pl.ds(base, K4C), pl.ds(0, 16)],
                                 pbb[p2], sems_p[p2])
                pltpu.async_copy(d4_hbm.at[pl.ds(base * 8, K4C * 8)],
                                 d4b[p2], sems_d[p2])

            @pl.when(cid == 1)
            def _():
                pltpu.async_copy(mhi_hbm.at[pl.ds(base, K4C), :],
                                 mb[p2], sems_m[p2])

            pltpu.async_copy(row_hbm.at[pl.ds(base, K4C)], rb[p2],
                             sems_r[p2])

        def _wait(g, p2):
            base = sid * per_tile + g * K4C
            pltpu.make_async_copy(mlo_hbm.at[pl.ds(base, K4C), :],
                                  mb[p2], sems_m[p2]).wait()
            pltpu.make_async_copy(row_hbm.at[pl.ds(base, K4C)], rb[p2],
                                  sems_r[p2]).wait()

            @pl.when(cid == 0)
            def _():
                pltpu.make_async_copy(pb_hbm.at[pl.ds(base, K4C), pl.ds(0, 16)],
                                      pbb[p2], sems_p[p2]).wait()
                pltpu.make_async_copy(d4_hbm.at[pl.ds(base * 8, K4C * 8)],
                                      d4b[p2], sems_d[p2]).wait()

        _issue(0, 0)

        def body(g2, _):
            for p2 in range(2):
                g = g2 * 2 + p2
                _wait(g, p2)
                if p2 == 0:
                    _issue(g + 1, 1)
                else:
                    @pl.when(g2 < n_chunks // 2 - 1)
                    def _():
                        _issue(g + 1, 0)
                for f in range(K4C // 16):
                    sl = pl.ds(f * 16, 16)
                    ri[sl] = rb[p2][sl]
                pltpu.sync_copy(mb[p2], accm.at[ri], add=True)

                @pl.when(cid == 0)
                def _():
                    zero_i = jnp.zeros((16,), jnp.int32)
                    for j in range(K4C // 16):
                        e16 = lane + (j * 16)
                        r16 = rb[p2][pl.ds(j * 16, 16)]
                        pri[pl.ds(j * 16, 16)] = r16 >> 4
                        ln = (r16 & 15) * 8
                        p16 = plsc.load_gather(pbb[p2], [e16, zero_i])
                        for c in range(3):
                            d16 = plsc.load_gather(d4b[p2], [e16 * 8 + c])
                            plsc.store_scatter(upd, [e16, ln + c], p16 * d16)
                        plsc.store_scatter(upd, [e16, ln + 3], one16)
                    pltpu.sync_copy(upd, accp.at[pri], add=True)
                    for j in range(K4C // 16):
                        e16 = lane + (j * 16)
                        r16 = rb[p2][pl.ds(j * 16, 16)]
                        ln = (r16 & 15) * 8
                        for c in range(4):
                            plsc.store_scatter(upd, [e16, ln + c], zero16)
            return 0

        lax.fori_loop(0, n_chunks // 2, body, 0)
        plsc.subcore_barrier()

        @pl.when(cid == 0)
        def _():
            pltpu.sync_copy(accm.at[rall, :], out0_hbm.at[rall, :])
            psl = pl.ds(sid * prow_per_tile, prow_per_tile)
            pltpu.sync_copy(accp.at[psl, :], outp_hbm.at[psl, :])

        @pl.when(cid == 1)
        def _():
            pltpu.sync_copy(accm.at[rall, :], out1_hbm.at[rall, :])

    return k4


def _node_post_kernel(nf_ref, a0_ref, a1_ref, ap_ref, posp_ref, velp_ref,
                      nfw1_ref, nfb1_ref, nfw2_ref, nfb2_ref,
                      vw1_ref, vb1_ref, vw2_ref, vb2_ref,
                      newf_ref, newp_ref):
    bf = jnp.bfloat16
    bn = nf_ref.shape[0]
    nf = nf_ref[...]
    apk = ap_ref[...]
    io_r = lax.broadcasted_iota(jnp.int32, (bn, bn // 16), 0) // 16
    io_c = lax.broadcasted_iota(jnp.int32, (bn, bn // 16), 1)
    sel = (io_r == io_c).astype(bf)
    rep = jnp.dot(sel, apk.astype(bf), preferred_element_type=jnp.float32)
    li = lax.broadcasted_iota(jnp.int32, (bn, HH), 1)
    ni = lax.broadcasted_iota(jnp.int32, (bn, HH), 0)
    cmask = (li == (ni % 16) * 8 + 3).astype(jnp.float32)
    ones_col = jnp.ones((HH, 1), bf)
    cnt_col = jnp.dot((rep * cmask).astype(bf), ones_col,
                      preferred_element_type=jnp.float32)
    cnt = jnp.maximum(cnt_col, 1.0)

    magg = jnp.concatenate([a0_ref[...], a1_ref[...]], axis=1) / cnt
    nfw = nfw1_ref[...].astype(bf)
    nf2_w1 = (jnp.dot(nf.astype(bf), nfw[0], preferred_element_type=jnp.float32)
              + jnp.dot(magg.astype(bf), nfw[1],
                        preferred_element_type=jnp.float32))
    hh = _silu(nf2_w1 + nfb1_ref[...])
    newf_ref[...] = jnp.dot(hh.astype(bf), nfw2_ref[...].astype(bf),
                            preferred_element_type=jnp.float32) + nfb2_ref[...]

    vh = _silu(jnp.dot(nf.astype(bf), vw1_ref[...].astype(bf),
                       preferred_element_type=jnp.float32) + vb1_ref[...])
    vf = jnp.dot(vh.astype(bf), vw2_ref[...].astype(bf),
                 preferred_element_type=jnp.float32) + vb2_ref[0, 0]
    tmask = (li // 8 == ni % 16).astype(jnp.float32)
    selT = (io_r == io_c).astype(bf).T
    vf_p = jnp.dot(selT, (vf * tmask).astype(bf),
                   preferred_element_type=jnp.float32)
    inv_p = jnp.dot(selT, (tmask / cnt).astype(bf),
                    preferred_element_type=jnp.float32)
    newp_ref[...] = posp_ref[...] + apk * inv_p + vf_p * velp_ref[...]


def _node_post(nf, a0, a1, accp, posp, velp, nfW1, nfb1, nfW2, nfb2,
               vW1, vb1, vW2, vb2, bn):
    n = nf.shape[0]
    grid = n // bn
    nfW1s = nfW1.reshape(2, H, H)
    return pl.pallas_call(
        _node_post_kernel,
        grid=(grid,),
        in_specs=[
            pl.BlockSpec((bn, H), lambda i: (i, 0)),
            pl.BlockSpec((bn, HH), lambda i: (i, 0)),
            pl.BlockSpec((bn, HH), lambda i: (i, 0)),
            pl.BlockSpec((bn // 16, HH), lambda i: (i, 0)),
            pl.BlockSpec((bn // 16, HH), lambda i: (i, 0)),
            pl.BlockSpec((bn // 16, HH), lambda i: (i, 0)),
            pl.BlockSpec((2, H, H), lambda i: (0, 0, 0)),
            pl.BlockSpec((1, H), lambda i: (0, 0)),
            pl.BlockSpec((H, H), lambda i: (0, 0)),
            pl.BlockSpec((1, H), lambda i: (0, 0)),
            pl.BlockSpec((H, H), lambda i: (0, 0)),
            pl.BlockSpec((1, H), lambda i: (0, 0)),
            pl.BlockSpec((H, 1), lambda i: (0, 0)),
            pl.BlockSpec((1, 1), lambda i: (0, 0), memory_space=pltpu.SMEM),
        ],
        out_specs=[
            pl.BlockSpec((bn, H), lambda i: (i, 0)),
            pl.BlockSpec((bn // 16, HH), lambda i: (i, 0)),
        ],
        out_shape=[
            jax.ShapeDtypeStruct((n, H), jnp.float32),
            jax.ShapeDtypeStruct((n // 16, HH), jnp.float32),
        ],
    )(nf, a0, a1, accp, posp, velp, nfW1s, nfb1, nfW2, nfb2,
      vW1, vb1, vW2, vb2)


def kernel(node_feat, node_pos, node_vel, edge_index, edge_attr, msg_W1,
           msg_b1, msg_W2, msg_b2, pos_W1, pos_b1, pos_W2, pos_b2, nf_W1,
           nf_b1, nf_W2, nf_b2, vel_W1, vel_b1, vel_W2, vel_b2):
    n = node_feat.shape[0]
    e = edge_index.shape[1]
    row = edge_index[0]
    col = edge_index[1]

    bn = 256
    n_pad = ((n + bn - 1) // bn) * bn
    step = NTILES * CHUNK
    e_pad = ((e + step - 1) // step) * step
    nfp = jnp.pad(node_feat, ((0, n_pad - n), (0, 0)))
    rowp = jnp.concatenate([row, jnp.full((e_pad - e,), n, jnp.int32)])
    colp = jnp.concatenate([col, jnp.full((e_pad - e,), n, jnp.int32)])

    w12 = jnp.stack([msg_W1[:H], msg_W1[H:2 * H]])
    alo, ahi, blo, bhi = _node_pre(nfp, w12, msg_b1[None, :], bn)

    posp = jnp.pad(node_pos, ((0, n_pad - n), (0, 0)))
    k2 = _make_gather_kernel(n_pad, e_pad)
    slo, shi, d4f, distE = k2(alo, ahi, blo, bhi, rowp, colp,
                              posp[:, 0], posp[:, 1], posp[:, 2])

    eaT = jnp.concatenate([
        jnp.pad(edge_attr.T, ((0, 0), (0, e_pad - e))),
        distE[None, :],
        jnp.zeros((7, e_pad), jnp.float32),
    ])
    w1c2 = jnp.concatenate([msg_W1[2 * H:2 * H + 17],
                            jnp.zeros((7, H), jnp.float32)])
    mlo, mhi, p_b = _edge_mlp(slo, shi, eaT, w1c2,
                              msg_W2, msg_b2[None, :], pos_W1,
                              pos_b1[None, :], pos_W2, pos_b2.reshape(1, 1))

    k4 = _make_scatter_kernel(n_pad, e_pad)
    zeros_acc = jnp.zeros((n_pad, HH), jnp.float32)
    a0, a1, accp = k4(mlo, mhi, p_b, d4f, rowp, zeros_acc)

    pos8 = jnp.pad(node_pos, ((0, n_pad - n), (0, 5))).reshape(n_pad // 16, HH)
    vel8 = jnp.pad(node_vel, ((0, n_pad - n), (0, 5))).reshape(n_pad // 16, HH)
    newf, newpP = _node_post(nfp, a0, a1, accp, pos8, vel8,
                             nf_W1, nf_b1[None, :], nf_W2, nf_b2[None, :],
                             vel_W1, vel_b1[None, :], vel_W2,
                             vel_b2.reshape(1, 1), bn)
    newp = newpP.reshape(n_pad, 8)[:n, :3]
    return (newf[:n], newp)
